# baseline (speedup 1.0000x reference)
import functools
import jax
import jax.numpy as jnp
from jax import lax
from jax.experimental import pallas as pl
from jax.experimental.pallas import tpu as pltpu

NUM_GENES = 50000
NUM_EDGES = 10000
NNZ = 500000
EMBED = 128
LATENT = 128
NUM_TREAT = 100
NUM_PATH = 50
P_PAD = 64
B = 1024
M = 200

TB = 32


def _dense_body(xg_ref, pmask_ref, ctx_ref,
                w1a_ref, w1b_ref, b1_ref, w2_ref, b2_ref,
                lw_ref, lb_ref, rw_ref, rb_ref,
                risk_ref, z_ref):
    ctx = ctx_ref[...]
    ctx_h = jnp.dot(ctx, w1b_ref[...], preferred_element_type=jnp.float32)

    def one_batch(b):
        xg = xg_ref[b]
        pm = pmask_ref[b]
        pgs = lax.dot_general(pm, xg, (((0,), (0,)), ((), ())),
                              preferred_element_type=jnp.float32)
        counts = jnp.clip(jnp.sum(pm, axis=0), 1.0, None)
        reps = pgs / counts[:, None]
        h = jnp.tanh(jnp.dot(reps, w1a_ref[...],
                             preferred_element_type=jnp.float32)
                     + ctx_h[b][None, :] + b1_ref[...])
        scores = jnp.dot(h, w2_ref[...],
                         preferred_element_type=jnp.float32)[:, 0] + b2_ref[0, 0]
        pid = lax.broadcasted_iota(jnp.int32, (P_PAD,), 0)
        scores = jnp.where(pid < NUM_PATH, scores, -jnp.inf)
        scores = scores - jnp.max(scores)
        e = jnp.exp(scores)
        w = e / jnp.sum(e)
        z = jnp.dot(w[None, :], reps,
                    preferred_element_type=jnp.float32)
        z_ref[b, :] = z[0]

    for b in range(TB):
        one_batch(b)
    zlat = (jnp.dot(z_ref[...], lw_ref[...], preferred_element_type=jnp.float32)
            + lb_ref[...])
    z_ref[...] = zlat
    risk_ref[...] = (jnp.dot(zlat, rw_ref[...],
                             preferred_element_type=jnp.float32)
                     + rb_ref[0, 0])


def _dense_stage(xg, pmask, ctx, path_w1, path_b1, path_w2, path_b2,
                 latent_w, latent_b, risk_w, risk_b):
    w1a = path_w1[:EMBED]
    w1b = path_w1[EMBED:]
    grid = (B // TB,)
    flt = jnp.float32
    risk, z = pl.pallas_call(
        _dense_body,
        grid=grid,
        in_specs=[
            pl.BlockSpec((TB, M, EMBED), lambda i: (i, 0, 0)),
            pl.BlockSpec((TB, M, P_PAD), lambda i: (i, 0, 0)),
            pl.BlockSpec((TB, EMBED), lambda i: (i, 0)),
            pl.BlockSpec((EMBED, EMBED), lambda i: (0, 0)),
            pl.BlockSpec((EMBED, EMBED), lambda i: (0, 0)),
            pl.BlockSpec((EMBED,), lambda i: (0,)),
            pl.BlockSpec((EMBED, 1), lambda i: (0, 0)),
            pl.BlockSpec((1, 1), lambda i: (0, 0)),
            pl.BlockSpec((EMBED, LATENT), lambda i: (0, 0)),
            pl.BlockSpec((LATENT,), lambda i: (0,)),
            pl.BlockSpec((LATENT, 1), lambda i: (0, 0)),
            pl.BlockSpec((1, 1), lambda i: (0, 0)),
        ],
        out_specs=[
            pl.BlockSpec((TB, 1), lambda i: (i, 0)),
            pl.BlockSpec((TB, LATENT), lambda i: (i, 0)),
        ],
        out_shape=[
            jax.ShapeDtypeStruct((B, 1), flt),
            jax.ShapeDtypeStruct((B, LATENT), flt),
        ],
    )(xg, pmask, ctx, w1a, w1b, path_b1, path_w2,
      path_b2.reshape(1, 1), latent_w, latent_b, risk_w, risk_b.reshape(1, 1))
    return risk[:, 0], z


def kernel(gene_ids, context_ids, gene_embed, treat_embed, h_rows, h_cols,
           h_vals, gene_pathway, path_w1, path_b1, path_w2, path_b2,
           latent_w, latent_b, risk_w, risk_b):
    Dv = jax.ops.segment_sum(h_vals, h_rows, num_segments=NUM_GENES)
    De = jax.ops.segment_sum(h_vals, h_cols, num_segments=NUM_EDGES)
    Dv_inv_sqrt = jnp.power(Dv + 1e-06, -0.5)[:, None]
    De_inv = jnp.power(De + 1e-06, -1.0)[:, None]
    X = gene_embed * Dv_inv_sqrt
    HX = jax.ops.segment_sum(X[h_rows], h_cols, num_segments=NUM_EDGES)
    HX = HX * De_inv
    X_prop = jax.ops.segment_sum(HX[h_cols], h_rows, num_segments=NUM_GENES)
    X_prop = X_prop * Dv_inv_sqrt

    xg = X_prop[gene_ids]
    ctx = treat_embed[context_ids]
    pmask = gene_pathway[gene_ids]
    pmask = jnp.pad(pmask, ((0, 0), (0, 0), (0, P_PAD - NUM_PATH)))

    return _dense_stage(xg, pmask, ctx, path_w1, path_b1, path_w2, path_b2,
                        latent_w, latent_b, risk_w, risk_b)


# speedup vs baseline: 1.3805x; 1.3805x over previous
import functools
import jax
import jax.numpy as jnp
from jax import lax
from jax.experimental import pallas as pl
from jax.experimental.pallas import tpu as pltpu
from jax.experimental.pallas import tpu_sc as plsc

NUM_GENES = 50000
NUM_EDGES = 10000
NNZ = 500000
EMBED = 128
LATENT = 128
NUM_TREAT = 100
NUM_PATH = 50
P_PAD = 64
B = 1024
M = 200

TB = 32

NC = 2
NS = 16
NW = NC * NS
PER_TILE = NNZ // NW
NCH = 124
TILE_PAD = NCH * 128 - PER_TILE
N_DUMP = 16
EACC = 10240


def _hop1_body(x_hbm, rows_hbm, cols_hbm, out_hbm,
               rows_v, cols_v, buf0, hx_sh, sem):
    cid = lax.axis_index("c")
    tid = lax.axis_index("s")
    wid = tid * NC + cid

    pltpu.sync_copy(rows_hbm.at[wid], rows_v)
    pltpu.sync_copy(cols_hbm.at[wid], cols_v)

    zeros = jnp.zeros((16,), jnp.float32)

    def zrow(r, _):
        for k in range(8):
            buf0[r, pl.ds(k * 16, 16)] = zeros
        return 0
    lax.fori_loop(0, 128, zrow, 0)

    base = tid * (EACC // NS)
    for s in range(5):
        pltpu.sync_copy(buf0, hx_sh.at[pl.ds(base + s * 128, 128)])
    plsc.subcore_barrier()

    def chunk(j, _):
        pltpu.async_copy(x_hbm.at[rows_v.at[j]], buf0, sem).wait()
        pltpu.sync_copy(buf0, hx_sh.at[cols_v.at[j]], add=True)
        return 0
    lax.fori_loop(0, NCH, chunk, 0)

    plsc.subcore_barrier()
    wbase = tid * (EACC // NS)
    pltpu.sync_copy(hx_sh.at[pl.ds(wbase, EACC // NS)],
                    out_hbm.at[cid, pl.ds(wbase, EACC // NS)])


def _hop1(x, rows_t, cols_t):
    mesh = plsc.VectorSubcoreMesh(core_axis_name="c", subcore_axis_name="s")
    return pl.kernel(
        _hop1_body,
        out_type=jax.ShapeDtypeStruct((NC, EACC, EMBED), jnp.float32),
        mesh=mesh,
        scratch_types=[
            pltpu.VMEM((NCH, 128), jnp.int32),
            pltpu.VMEM((NCH, 128), jnp.int32),
            pltpu.VMEM((128, EMBED), jnp.float32),
            pltpu.VMEM_SHARED((EACC, EMBED), jnp.float32),
            pltpu.SemaphoreType.DMA,
        ],
    )(x, rows_t, cols_t)


def _pad_pairs(h_rows, h_cols, dump_base):
    pad_g = (jnp.arange(TILE_PAD, dtype=jnp.int32) * 97) % NUM_GENES
    pad_g = jnp.broadcast_to(pad_g[None, :], (NW, TILE_PAD))
    pad_s = dump_base + (jnp.arange(TILE_PAD, dtype=jnp.int32) % N_DUMP)
    pad_s = jnp.broadcast_to(pad_s[None, :], (NW, TILE_PAD))
    rows_t = jnp.concatenate(
        [h_rows.reshape(NW, PER_TILE), pad_g], axis=1).reshape(NW, NCH, 128)
    cols_t = jnp.concatenate(
        [h_cols.reshape(NW, PER_TILE), pad_s], axis=1).reshape(NW, NCH, 128)
    return rows_t, cols_t


def _dense_body(xg_ref, pmask_ref, ctx_ref,
                w1a_ref, w1b_ref, b1_ref, w2_ref, b2_ref,
                lw_ref, lb_ref, rw_ref, rb_ref,
                risk_ref, z_ref):
    ctx = ctx_ref[...]
    ctx_h = jnp.dot(ctx, w1b_ref[...], preferred_element_type=jnp.float32)

    def one_batch(b):
        xg = xg_ref[b]
        pm = pmask_ref[b]
        pgs = lax.dot_general(pm, xg, (((0,), (0,)), ((), ())),
                              preferred_element_type=jnp.float32)
        counts = jnp.clip(jnp.sum(pm, axis=0), 1.0, None)
        reps = pgs / counts[:, None]
        h = jnp.tanh(jnp.dot(reps, w1a_ref[...],
                             preferred_element_type=jnp.float32)
                     + ctx_h[b][None, :] + b1_ref[...])
        scores = jnp.dot(h, w2_ref[...],
                         preferred_element_type=jnp.float32)[:, 0] + b2_ref[0, 0]
        pid = lax.broadcasted_iota(jnp.int32, (P_PAD,), 0)
        scores = jnp.where(pid < NUM_PATH, scores, -jnp.inf)
        scores = scores - jnp.max(scores)
        e = jnp.exp(scores)
        w = e / jnp.sum(e)
        z = jnp.dot(w[None, :], reps,
                    preferred_element_type=jnp.float32)
        z_ref[b, :] = z[0]

    for b in range(TB):
        one_batch(b)
    zlat = (jnp.dot(z_ref[...], lw_ref[...], preferred_element_type=jnp.float32)
            + lb_ref[...])
    z_ref[...] = zlat
    risk_ref[...] = (jnp.dot(zlat, rw_ref[...],
                             preferred_element_type=jnp.float32)
                     + rb_ref[0, 0])


def _dense_stage(xg, pmask, ctx, path_w1, path_b1, path_w2, path_b2,
                 latent_w, latent_b, risk_w, risk_b):
    w1a = path_w1[:EMBED]
    w1b = path_w1[EMBED:]
    grid = (B // TB,)
    flt = jnp.float32
    risk, z = pl.pallas_call(
        _dense_body,
        grid=grid,
        in_specs=[
            pl.BlockSpec((TB, M, EMBED), lambda i: (i, 0, 0)),
            pl.BlockSpec((TB, M, P_PAD), lambda i: (i, 0, 0)),
            pl.BlockSpec((TB, EMBED), lambda i: (i, 0)),
            pl.BlockSpec((EMBED, EMBED), lambda i: (0, 0)),
            pl.BlockSpec((EMBED, EMBED), lambda i: (0, 0)),
            pl.BlockSpec((EMBED,), lambda i: (0,)),
            pl.BlockSpec((EMBED, 1), lambda i: (0, 0)),
            pl.BlockSpec((1, 1), lambda i: (0, 0)),
            pl.BlockSpec((EMBED, LATENT), lambda i: (0, 0)),
            pl.BlockSpec((LATENT,), lambda i: (0,)),
            pl.BlockSpec((LATENT, 1), lambda i: (0, 0)),
            pl.BlockSpec((1, 1), lambda i: (0, 0)),
        ],
        out_specs=[
            pl.BlockSpec((TB, 1), lambda i: (i, 0)),
            pl.BlockSpec((TB, LATENT), lambda i: (i, 0)),
        ],
        out_shape=[
            jax.ShapeDtypeStruct((B, 1), flt),
            jax.ShapeDtypeStruct((B, LATENT), flt),
        ],
    )(xg, pmask, ctx, w1a, w1b, path_b1, path_w2,
      path_b2.reshape(1, 1), latent_w, latent_b, risk_w, risk_b.reshape(1, 1))
    return risk[:, 0], z


def kernel(gene_ids, context_ids, gene_embed, treat_embed, h_rows, h_cols,
           h_vals, gene_pathway, path_w1, path_b1, path_w2, path_b2,
           latent_w, latent_b, risk_w, risk_b):
    Dv = jax.ops.segment_sum(h_vals, h_rows, num_segments=NUM_GENES)
    De = jax.ops.segment_sum(h_vals, h_cols, num_segments=NUM_EDGES)
    Dv_inv_sqrt = jnp.power(Dv + 1e-06, -0.5)[:, None]
    De_inv = jnp.power(De + 1e-06, -1.0)[:, None]
    X = gene_embed * Dv_inv_sqrt
    rows_t, cols_t = _pad_pairs(h_rows, h_cols, NUM_EDGES)
    hx_part = _hop1(X, rows_t, cols_t)
    HX = (hx_part[0, :NUM_EDGES] + hx_part[1, :NUM_EDGES]) * De_inv
    X_prop = jax.ops.segment_sum(HX[h_cols], h_rows, num_segments=NUM_GENES)
    X_prop = X_prop * Dv_inv_sqrt

    xg = X_prop[gene_ids]
    ctx = treat_embed[context_ids]
    pmask = gene_pathway[gene_ids]
    pmask = jnp.pad(pmask, ((0, 0), (0, 0), (0, P_PAD - NUM_PATH)))

    return _dense_stage(xg, pmask, ctx, path_w1, path_b1, path_w2, path_b2,
                        latent_w, latent_b, risk_w, risk_b)
